# baseline (speedup 1.0000x reference)
import jax
import jax.numpy as jnp
from jax.experimental import pallas as pl


def _mlp_body(h_ref, w1_ref, b1_ref, w2_ref, b2_ref, o_ref):
    h = h_ref[...]
    t = jnp.dot(h, w1_ref[...], preferred_element_type=jnp.float32)
    t = jnp.maximum(t + b1_ref[...], 0.0)
    o = jnp.dot(t, w2_ref[...], preferred_element_type=jnp.float32)
    o_ref[...] = jnp.maximum(o + b2_ref[...], 0.0)


def _out_body(h_ref, w_ref, b_ref, o_ref):
    h = h_ref[...]
    t = jnp.dot(h, w_ref[...], preferred_element_type=jnp.float32)
    o_ref[...] = jax.nn.sigmoid(t + b_ref[...])


def _tc_mlp(h, W1, b1, W2, b2, bn=2000):
    n, d_in = h.shape
    d_mid = W1.shape[1]
    return pl.pallas_call(
        _mlp_body,
        grid=(n // bn,),
        in_specs=[
            pl.BlockSpec((bn, d_in), lambda i: (i, 0)),
            pl.BlockSpec((d_in, d_mid), lambda i: (0, 0)),
            pl.BlockSpec((1, d_mid), lambda i: (0, 0)),
            pl.BlockSpec((d_mid, d_mid), lambda i: (0, 0)),
            pl.BlockSpec((1, d_mid), lambda i: (0, 0)),
        ],
        out_specs=pl.BlockSpec((bn, d_mid), lambda i: (i, 0)),
        out_shape=jax.ShapeDtypeStruct((n, d_mid), jnp.float32),
    )(h, W1, b1.reshape(1, -1), W2, b2.reshape(1, -1))


def _tc_out(h, W, b, bn=2000):
    n, d_in = h.shape
    d_out = W.shape[1]
    return pl.pallas_call(
        _out_body,
        grid=(n // bn,),
        in_specs=[
            pl.BlockSpec((bn, d_in), lambda i: (i, 0)),
            pl.BlockSpec((d_in, d_out), lambda i: (0, 0)),
            pl.BlockSpec((1, d_out), lambda i: (0, 0)),
        ],
        out_specs=pl.BlockSpec((bn, d_out), lambda i: (i, 0)),
        out_shape=jax.ShapeDtypeStruct((n, d_out), jnp.float32),
    )(h, W, b.reshape(1, -1))


def kernel(x, edge_index, g1_W1, g1_b1, g1_W2, g1_b2, g2_W1, g2_b1, g2_W2, g2_b2, g3_W1, g3_b1, g3_W2, g3_b2, g4_W1, g4_b1, g4_W2, g4_b2, g5_W1, g5_b1, g5_W2, g5_b2, g6_W1, g6_b1, g6_W2, g6_b2, Wout, bout):
    src = edge_index[0]
    dst = edge_index[1]
    layers = [
        (g1_W1, g1_b1, g1_W2, g1_b2),
        (g2_W1, g2_b1, g2_W2, g2_b2),
        (g3_W1, g3_b1, g3_W2, g3_b2),
        (g4_W1, g4_b1, g4_W2, g4_b2),
        (g5_W1, g5_b1, g5_W2, g5_b2),
        (g6_W1, g6_b1, g6_W2, g6_b2),
    ]
    for W1, b1, W2, b2 in layers:
        agg = jax.ops.segment_sum(x[src], dst, num_segments=x.shape[0])
        x = _tc_mlp(x + agg, W1, b1, W2, b2)
    return _tc_out(x, Wout, bout)
